# baseline (speedup 1.0000x reference)
import functools

import jax
import jax.numpy as jnp
from jax import lax
from jax.experimental import pallas as pl
from jax.experimental.pallas import tpu as pltpu
from jax.experimental.pallas import tpu_sc as plsc

N = 10000
D = 128
E = 320000

NC = 2
NS = 16
NW = NC * NS

EPW = E // NW
EB = 128
SEG = 40
NSEG = 2

N_PAD = 10240
RPT = N_PAD // NS
RBLK = 1280
GRID = N_PAD // RBLK

_mesh = plsc.VectorSubcoreMesh(core_axis_name="c", subcore_axis_name="s")


@functools.partial(
    pl.kernel,
    out_type=jax.ShapeDtypeStruct((NC, 2, N_PAD), jnp.float32),
    mesh=_mesh,
    scratch_types=[
        pltpu.VMEM((SEG + 2, EB), jnp.int32),
        pltpu.VMEM((SEG, EB), jnp.int32),
        pltpu.VMEM((EB,), jnp.float32),
        pltpu.VMEM_SHARED((N_PAD,), jnp.float32),
        pltpu.VMEM_SHARED((N_PAD,), jnp.float32),
        pltpu.SemaphoreType.DMA,
    ],
)
def _deg_kernel(si, di, z1, deg_out, idx_s, idx_d, ones_v, dsp_s, dsp_d,
                sem_s):
    c = lax.axis_index("c")
    s = lax.axis_index("s")
    wid = c * NS + s

    @pl.when(s == 0)
    def _():
        pltpu.sync_copy(z1, dsp_s)

    @pl.when(s == 1)
    def _():
        pltpu.sync_copy(z1, dsp_d)

    for k in range(EB // 16):
        ones_v[pl.ds(16 * k, 16)] = jnp.full((16,), 1.0, jnp.float32)

    plsc.subcore_barrier()

    def body(j, carry):
        pltpu.async_copy(ones_v, dsp_s.at[idx_s.at[j]], sem_s, add=True)
        pltpu.async_copy(ones_v, dsp_d.at[idx_d.at[j]], sem_s, add=True)
        return carry

    def drain(j, carry):
        pltpu.make_async_copy(ones_v, dsp_s.at[idx_s.at[0]], sem_s).wait()
        pltpu.make_async_copy(ones_v, dsp_d.at[idx_d.at[0]], sem_s).wait()
        return carry

    for seg in range(NSEG):
        pltpu.sync_copy(si.at[wid, seg], idx_s)
        pltpu.sync_copy(di.at[wid, seg], idx_d)
        lax.fori_loop(0, SEG, body, 0)
        lax.fori_loop(0, SEG, drain, 0)
    plsc.subcore_barrier()

    @pl.when(s == 0)
    def _():
        pltpu.sync_copy(dsp_s, deg_out.at[c, 0])

    @pl.when(s == 1)
    def _():
        pltpu.sync_copy(dsp_d, deg_out.at[c, 1])


@functools.partial(
    pl.kernel,
    out_type=jax.ShapeDtypeStruct((NC, N_PAD, D), jnp.float32),
    mesh=_mesh,
    scratch_types=[
        pltpu.VMEM((SEG + 2, EB), jnp.int32),
        pltpu.VMEM((SEG, EB), jnp.int32),
        pltpu.VMEM((EB, D), jnp.float32),
        pltpu.VMEM((EB, D), jnp.float32),
        pltpu.VMEM_SHARED((N_PAD, D), jnp.float32),
        pltpu.SemaphoreType.DMA,
        pltpu.SemaphoreType.DMA,
    ],
)
def _agg_kernel(h, si, di, z2, out, idx_s, idx_d, rows_a, rows_b, acc,
                sem_a, sem_s):
    c = lax.axis_index("c")
    s = lax.axis_index("s")
    wid = c * NS + s
    hc = h.at[c * 8 + s // 2]

    pltpu.sync_copy(z2.at[pl.ds(s * RPT, RPT)], acc.at[pl.ds(s * RPT, RPT)])
    plsc.subcore_barrier()

    def body(t, carry):
        jj = 2 * t

        @pl.when(t > 0)
        def _():
            pltpu.make_async_copy(rows_a, acc.at[idx_d.at[0]], sem_s).wait()

        pltpu.async_copy(hc.at[idx_s.at[jj]], rows_a, sem_a).wait()
        pltpu.async_copy(rows_a, acc.at[idx_d.at[jj]], sem_s, add=True)

        @pl.when(t > 0)
        def _():
            pltpu.make_async_copy(rows_b, acc.at[idx_d.at[0]], sem_s).wait()

        pltpu.async_copy(hc.at[idx_s.at[jj + 1]], rows_b, sem_a).wait()
        pltpu.async_copy(rows_b, acc.at[idx_d.at[jj + 1]], sem_s, add=True)
        return carry

    for seg in range(NSEG):
        pltpu.sync_copy(si.at[wid, seg], idx_s)
        pltpu.sync_copy(di.at[wid, seg], idx_d)
        lax.fori_loop(0, SEG // 2, body, 0)
        pltpu.make_async_copy(rows_a, acc.at[idx_d.at[0]], sem_s).wait()
        pltpu.make_async_copy(rows_b, acc.at[idx_d.at[0]], sem_s).wait()

    plsc.subcore_barrier()
    pltpu.sync_copy(acc.at[pl.ds(s * RPT, RPT)], out.at[c, pl.ds(s * RPT, RPT)])


def _mm1_body(x_ref, w_ref, d_ref, o_ref):
    deg_s = d_ref[0, 0] + d_ref[1, 0]
    ns = lax.rsqrt(jnp.maximum(deg_s, 1.0))
    r = jnp.dot(
        x_ref[...], w_ref[...], preferred_element_type=jnp.float32
    ) * ns[:, None]
    for k in range(16):
        o_ref[k] = r


def _mm2_body(p_ref, d_ref, b_ref, w_ref, o_ref):
    agg = p_ref[0] + p_ref[1]
    nd = lax.rsqrt(jnp.maximum(d_ref[0, 1] + d_ref[1, 1], 1.0))
    ns = lax.rsqrt(jnp.maximum(d_ref[0, 0] + d_ref[1, 0], 1.0))
    h = jnp.maximum(agg * nd[:, None] + b_ref[0][None, :], 0.0)
    r = jnp.dot(
        h, w_ref[...], preferred_element_type=jnp.float32
    ) * ns[:, None]
    for k in range(16):
        o_ref[k] = r


def _fin_body(p_ref, d_ref, b_ref, o_ref):
    agg = p_ref[0] + p_ref[1]
    nd = lax.rsqrt(jnp.maximum(d_ref[0, 1] + d_ref[1, 1], 1.0))
    o_ref[...] = agg * nd[:, None] + b_ref[0][None, :]


def _mm1(x_p, W1, degs):
    return pl.pallas_call(
        _mm1_body,
        grid=(GRID,),
        in_specs=[
            pl.BlockSpec((RBLK, D), lambda i: (i, 0)),
            pl.BlockSpec((D, D), lambda i: (0, 0)),
            pl.BlockSpec((NC, 2, RBLK), lambda i: (0, 0, i)),
        ],
        out_specs=pl.BlockSpec((16, RBLK, D), lambda i: (0, i, 0)),
        out_shape=jax.ShapeDtypeStruct((16, N_PAD, D), jnp.float32),
    )(x_p, W1, degs)


def _mm2(p, degs, b1, W2):
    return pl.pallas_call(
        _mm2_body,
        grid=(GRID,),
        in_specs=[
            pl.BlockSpec((NC, RBLK, D), lambda i: (0, i, 0)),
            pl.BlockSpec((NC, 2, RBLK), lambda i: (0, 0, i)),
            pl.BlockSpec((1, D), lambda i: (0, 0)),
            pl.BlockSpec((D, D), lambda i: (0, 0)),
        ],
        out_specs=pl.BlockSpec((16, RBLK, D), lambda i: (0, i, 0)),
        out_shape=jax.ShapeDtypeStruct((16, N_PAD, D), jnp.float32),
    )(p, degs, b1, W2)


def _fin(p, degs, b2):
    return pl.pallas_call(
        _fin_body,
        grid=(GRID,),
        in_specs=[
            pl.BlockSpec((NC, RBLK, D), lambda i: (0, i, 0)),
            pl.BlockSpec((NC, 2, RBLK), lambda i: (0, 0, i)),
            pl.BlockSpec((1, D), lambda i: (0, 0)),
        ],
        out_specs=pl.BlockSpec((RBLK, D), lambda i: (i, 0)),
        out_shape=jax.ShapeDtypeStruct((N_PAD, D), jnp.float32),
    )(p, degs, b2)


def kernel(x, edge_index, W1, b1, W2, b2):
    e2 = edge_index.reshape(2, NW, EPW)
    epad = jnp.full((NW, NSEG * SEG * EB - EPW), N, jnp.int32)
    si = jnp.concatenate(
        [jnp.concatenate([e2[0], epad], axis=1).reshape(NW, NSEG, SEG, EB),
         jnp.zeros((NW, NSEG, 2, EB), jnp.int32)], axis=2)
    di = jnp.concatenate([e2[1], epad], axis=1).reshape(NW, NSEG, SEG, EB)

    x_p = jnp.concatenate(
        [x, jnp.zeros((N_PAD - N, D), jnp.float32)], axis=0)
    z1 = jnp.zeros((N_PAD,), jnp.float32)
    z2 = jnp.zeros((N_PAD, D), jnp.float32)
    b1r = b1.reshape(1, D)
    b2r = b2.reshape(1, D)

    degs = _deg_kernel(si, di, z1)
    h1 = _mm1(x_p, W1, degs)
    p1 = _agg_kernel(h1, si, di, z2)
    h2 = _mm2(p1, degs, b1r, W2)
    p2 = _agg_kernel(h2, si, di, z2)
    out = _fin(p2, degs, b2r)
    return out[:N]


# speedup vs baseline: 1.0563x; 1.0563x over previous
import functools

import jax
import jax.numpy as jnp
from jax import lax
from jax.experimental import pallas as pl
from jax.experimental.pallas import tpu as pltpu
from jax.experimental.pallas import tpu_sc as plsc

N = 10000
D = 128
E = 320000

NC = 2
NS = 16
NW = NC * NS

EPW = E // NW
EB = 128
SEG = 40
NSEG = 2

N_PAD = 10240
RPT = N_PAD // NS
RBLK = 1280
GRID = N_PAD // RBLK

_mesh = plsc.VectorSubcoreMesh(core_axis_name="c", subcore_axis_name="s")


@functools.partial(
    pl.kernel,
    out_type=jax.ShapeDtypeStruct((NC, 2, N_PAD), jnp.float32),
    mesh=_mesh,
    scratch_types=[
        pltpu.VMEM((SEG + 2, EB), jnp.int32),
        pltpu.VMEM((SEG, EB), jnp.int32),
        pltpu.VMEM((EB,), jnp.float32),
        pltpu.VMEM_SHARED((N_PAD,), jnp.float32),
        pltpu.VMEM_SHARED((N_PAD,), jnp.float32),
        pltpu.SemaphoreType.DMA,
    ],
)
def _deg_kernel(si, di, z1, deg_out, idx_s, idx_d, ones_v, dsp_s, dsp_d,
                sem_s):
    c = lax.axis_index("c")
    s = lax.axis_index("s")
    wid = c * NS + s

    @pl.when(s == 0)
    def _():
        pltpu.sync_copy(z1, dsp_s)

    @pl.when(s == 1)
    def _():
        pltpu.sync_copy(z1, dsp_d)

    for k in range(EB // 16):
        ones_v[pl.ds(16 * k, 16)] = jnp.full((16,), 1.0, jnp.float32)

    plsc.subcore_barrier()

    def body(j, carry):
        pltpu.async_copy(ones_v, dsp_s.at[idx_s.at[j]], sem_s, add=True)
        pltpu.async_copy(ones_v, dsp_d.at[idx_d.at[j]], sem_s, add=True)
        return carry

    def drain(j, carry):
        pltpu.make_async_copy(ones_v, dsp_s.at[idx_s.at[0]], sem_s).wait()
        pltpu.make_async_copy(ones_v, dsp_d.at[idx_d.at[0]], sem_s).wait()
        return carry

    for seg in range(NSEG):
        pltpu.sync_copy(si.at[wid, seg], idx_s)
        pltpu.sync_copy(di.at[wid, seg], idx_d)
        lax.fori_loop(0, SEG, body, 0)
        lax.fori_loop(0, SEG, drain, 0)
    plsc.subcore_barrier()

    @pl.when(s == 0)
    def _():
        pltpu.sync_copy(dsp_s, deg_out.at[c, 0])

    @pl.when(s == 1)
    def _():
        pltpu.sync_copy(dsp_d, deg_out.at[c, 1])


@functools.partial(
    pl.kernel,
    out_type=jax.ShapeDtypeStruct((NC, N_PAD, D), jnp.float32),
    mesh=_mesh,
    scratch_types=[
        pltpu.VMEM((SEG + 2, EB), jnp.int32),
        pltpu.VMEM((SEG, EB), jnp.int32),
        pltpu.VMEM((EB, D), jnp.float32),
        pltpu.VMEM((EB, D), jnp.float32),
        pltpu.VMEM_SHARED((N_PAD, D), jnp.float32),
        pltpu.SemaphoreType.DMA,
        pltpu.SemaphoreType.DMA,
    ],
)
def _agg_kernel(h, si, di, z2, out, idx_s, idx_d, rows_a, rows_b, acc,
                sem_a, sem_s):
    c = lax.axis_index("c")
    s = lax.axis_index("s")
    wid = c * NS + s
    hc = h.at[c * 4 + s // 4]

    pltpu.sync_copy(z2.at[pl.ds(s * RPT, RPT)], acc.at[pl.ds(s * RPT, RPT)])
    plsc.subcore_barrier()

    def body(t, carry):
        jj = 2 * t

        @pl.when(t > 0)
        def _():
            pltpu.make_async_copy(rows_a, acc.at[idx_d.at[0]], sem_s).wait()

        pltpu.async_copy(hc.at[idx_s.at[jj]], rows_a, sem_a).wait()
        pltpu.async_copy(rows_a, acc.at[idx_d.at[jj]], sem_s, add=True)

        @pl.when(t > 0)
        def _():
            pltpu.make_async_copy(rows_b, acc.at[idx_d.at[0]], sem_s).wait()

        pltpu.async_copy(hc.at[idx_s.at[jj + 1]], rows_b, sem_a).wait()
        pltpu.async_copy(rows_b, acc.at[idx_d.at[jj + 1]], sem_s, add=True)
        return carry

    for seg in range(NSEG):
        pltpu.sync_copy(si.at[wid, seg], idx_s)
        pltpu.sync_copy(di.at[wid, seg], idx_d)
        lax.fori_loop(0, SEG // 2, body, 0)
        pltpu.make_async_copy(rows_a, acc.at[idx_d.at[0]], sem_s).wait()
        pltpu.make_async_copy(rows_b, acc.at[idx_d.at[0]], sem_s).wait()

    plsc.subcore_barrier()
    pltpu.sync_copy(acc.at[pl.ds(s * RPT, RPT)], out.at[c, pl.ds(s * RPT, RPT)])


def _mm1_body(x_ref, w_ref, d_ref, o_ref):
    deg_s = d_ref[0, 0] + d_ref[1, 0]
    ns = lax.rsqrt(jnp.maximum(deg_s, 1.0))
    r = jnp.dot(
        x_ref[...], w_ref[...], preferred_element_type=jnp.float32
    ) * ns[:, None]
    for k in range(8):
        o_ref[k] = r


def _mm2_body(p_ref, d_ref, b_ref, w_ref, o_ref):
    agg = p_ref[0] + p_ref[1]
    nd = lax.rsqrt(jnp.maximum(d_ref[0, 1] + d_ref[1, 1], 1.0))
    ns = lax.rsqrt(jnp.maximum(d_ref[0, 0] + d_ref[1, 0], 1.0))
    h = jnp.maximum(agg * nd[:, None] + b_ref[0][None, :], 0.0)
    r = jnp.dot(
        h, w_ref[...], preferred_element_type=jnp.float32
    ) * ns[:, None]
    for k in range(8):
        o_ref[k] = r


def _fin_body(p_ref, d_ref, b_ref, o_ref):
    agg = p_ref[0] + p_ref[1]
    nd = lax.rsqrt(jnp.maximum(d_ref[0, 1] + d_ref[1, 1], 1.0))
    o_ref[...] = agg * nd[:, None] + b_ref[0][None, :]


def _mm1(x_p, W1, degs):
    return pl.pallas_call(
        _mm1_body,
        grid=(GRID,),
        in_specs=[
            pl.BlockSpec((RBLK, D), lambda i: (i, 0)),
            pl.BlockSpec((D, D), lambda i: (0, 0)),
            pl.BlockSpec((NC, 2, RBLK), lambda i: (0, 0, i)),
        ],
        out_specs=pl.BlockSpec((8, RBLK, D), lambda i: (0, i, 0)),
        out_shape=jax.ShapeDtypeStruct((8, N_PAD, D), jnp.float32),
    )(x_p, W1, degs)


def _mm2(p, degs, b1, W2):
    return pl.pallas_call(
        _mm2_body,
        grid=(GRID,),
        in_specs=[
            pl.BlockSpec((NC, RBLK, D), lambda i: (0, i, 0)),
            pl.BlockSpec((NC, 2, RBLK), lambda i: (0, 0, i)),
            pl.BlockSpec((1, D), lambda i: (0, 0)),
            pl.BlockSpec((D, D), lambda i: (0, 0)),
        ],
        out_specs=pl.BlockSpec((8, RBLK, D), lambda i: (0, i, 0)),
        out_shape=jax.ShapeDtypeStruct((8, N_PAD, D), jnp.float32),
    )(p, degs, b1, W2)


def _fin(p, degs, b2):
    return pl.pallas_call(
        _fin_body,
        grid=(GRID,),
        in_specs=[
            pl.BlockSpec((NC, RBLK, D), lambda i: (0, i, 0)),
            pl.BlockSpec((NC, 2, RBLK), lambda i: (0, 0, i)),
            pl.BlockSpec((1, D), lambda i: (0, 0)),
        ],
        out_specs=pl.BlockSpec((RBLK, D), lambda i: (i, 0)),
        out_shape=jax.ShapeDtypeStruct((N_PAD, D), jnp.float32),
    )(p, degs, b2)


def kernel(x, edge_index, W1, b1, W2, b2):
    e2 = edge_index.reshape(2, NW, EPW)
    epad = jnp.full((NW, NSEG * SEG * EB - EPW), N, jnp.int32)
    si = jnp.concatenate(
        [jnp.concatenate([e2[0], epad], axis=1).reshape(NW, NSEG, SEG, EB),
         jnp.zeros((NW, NSEG, 2, EB), jnp.int32)], axis=2)
    di = jnp.concatenate([e2[1], epad], axis=1).reshape(NW, NSEG, SEG, EB)

    x_p = jnp.concatenate(
        [x, jnp.zeros((N_PAD - N, D), jnp.float32)], axis=0)
    z1 = jnp.zeros((N_PAD,), jnp.float32)
    z2 = jnp.zeros((N_PAD, D), jnp.float32)
    b1r = b1.reshape(1, D)
    b2r = b2.reshape(1, D)

    degs = _deg_kernel(si, di, z1)
    h1 = _mm1(x_p, W1, degs)
    p1 = _agg_kernel(h1, si, di, z2)
    h2 = _mm2(p1, degs, b1r, W2)
    p2 = _agg_kernel(h2, si, di, z2)
    out = _fin(p2, degs, b2r)
    return out[:N]
